# baseline (device time: 27240 ns/iter reference)
import jax
import jax.numpy as jnp
from jax import lax
from jax.experimental import pallas as pl
from jax.experimental.pallas import tpu as pltpu

N_DEV = 16


def kernel(x, w_mat, scale_x, scale_w):
    m_total, k_loc = x.shape
    k_total, n_out = w_mat.shape
    m_per = m_total // N_DEV

    my_pos = lax.axis_index("i")
    steps = jnp.arange(N_DEV, dtype=jnp.int32)
    wtab = jnp.remainder(my_pos.astype(jnp.int32) - steps, N_DEV)

    def body(wtab_ref, x_ref, w_ref, sx_ref, sw_ref, out_ref,
             comm_ref, acc_ref, send_sems, recv_sems):
        s = pl.program_id(0)
        my = lax.axis_index("i")

        dims = (((1,), (0,)), ((), ()))

        @pl.when(s == 0)
        def _():
            barrier = pltpu.get_barrier_semaphore()
            for d in range(1, N_DEV):
                t = lax.rem(my + d, N_DEV)
                pl.semaphore_signal(barrier, inc=1, device_id=(t,),
                                    device_id_type=pl.DeviceIdType.MESH)
            pl.semaphore_wait(barrier, N_DEV - 1)

            for d in range(1, N_DEV):
                t = lax.rem(my + d, N_DEV)
                pltpu.make_async_remote_copy(
                    src_ref=x_ref.at[pl.ds(t * m_per, m_per), :],
                    dst_ref=comm_ref.at[d],
                    send_sem=send_sems.at[d],
                    recv_sem=recv_sems.at[d],
                    device_id=(t,),
                    device_id_type=pl.DeviceIdType.MESH,
                ).start()

            xa = x_ref[pl.ds(my * m_per, m_per), :]
            acc_ref[...] = lax.dot_general(
                xa, w_ref[...], dims, preferred_element_type=jnp.int32)

        @pl.when(s > 0)
        def _():
            rdma = pltpu.make_async_remote_copy(
                src_ref=x_ref.at[pl.ds(0, m_per), :],
                dst_ref=comm_ref.at[s],
                send_sem=send_sems.at[s],
                recv_sem=recv_sems.at[s],
                device_id=(my,),
                device_id_type=pl.DeviceIdType.MESH,
            )
            rdma.wait_recv()
            acc_ref[...] += lax.dot_general(
                comm_ref[s], w_ref[...], dims, preferred_element_type=jnp.int32)
            rdma.wait_send()

        @pl.when(s == N_DEV - 1)
        def _():
            alpha = sx_ref[0] * sw_ref[0]
            y = acc_ref[...].astype(jnp.float32) * alpha
            out_ref[...] = jnp.maximum(y, 0.0)

    grid_spec = pltpu.PrefetchScalarGridSpec(
        num_scalar_prefetch=1,
        grid=(N_DEV,),
        in_specs=[
            pl.BlockSpec((m_total, k_loc), lambda s, wt: (0, 0)),
            pl.BlockSpec((k_total // N_DEV, n_out),
                         lambda s, wt: (wt[s], 0)),
            pl.BlockSpec(memory_space=pltpu.SMEM),
            pl.BlockSpec(memory_space=pltpu.SMEM),
        ],
        out_specs=pl.BlockSpec((m_per, n_out), lambda s, wt: (0, 0)),
        scratch_shapes=[
            pltpu.VMEM((N_DEV, m_per, k_loc), jnp.int8),
            pltpu.VMEM((m_per, n_out), jnp.int32),
            pltpu.SemaphoreType.DMA((N_DEV,)),
            pltpu.SemaphoreType.DMA((N_DEV,)),
        ],
    )

    return pl.pallas_call(
        body,
        grid_spec=grid_spec,
        out_shape=jax.ShapeDtypeStruct((m_per, n_out), jnp.float32),
        compiler_params=pltpu.CompilerParams(
            collective_id=0,
            dimension_semantics=("arbitrary",),
        ),
    )(wtab, x, w_mat, scale_x, scale_w)
